# Initial kernel scaffold; baseline (speedup 1.0000x reference)
import jax
import jax.numpy as jnp
from jax.experimental import pallas as pl


def kernel(x, edge_index, W_ig, b_ig, W_og, b_og, W_in, b_in, W_gw, b_gw, W_out, b_out, arma_w, arma_root, arma_bias):
    raise NotImplementedError("write your pallas kernel here")


# speedup vs baseline: 16.5432x; 16.5432x over previous
import functools

import jax
import jax.numpy as jnp
from jax import lax
from jax.experimental import pallas as pl
from jax.experimental.pallas import tpu as pltpu
from jax.experimental.pallas import tpu_sc as plsc


def _gelu(v):
    return 0.5 * v * (1.0 + lax.erf(v * 0.7071067811865476))


def _sigmoid(v):
    return 1.0 / (1.0 + jnp.exp(-v))


def _dot_t(a, b):
    return lax.dot_general(a, b, (((1,), (1,)), ((), ())),
                           preferred_element_type=jnp.float32)


def _dot(a, b):
    return lax.dot_general(a, b, (((1,), (0,)), ((), ())),
                           preferred_element_type=jnp.float32)


def _tc_pre(x, W_ig, b_ig, W_og, b_og, W_in, b_in, arma_w, arma_root):
    N, C = x.shape
    BM = 1000

    def body(x_r, wig_r, big_r, wog_r, bog_r, win_r, bin_r, aw_r, ar_r,
             og_r, hr_r, tp_r):
        xb = x_r[...]
        ig = _sigmoid(_dot_t(xb, wig_r[...]) + big_r[...])
        og_r[...] = _sigmoid(_dot_t(xb, wog_r[...]) + bog_r[...])
        h = _gelu(_dot_t(ig * xb, win_r[...]) + bin_r[...])
        tp_r[...] = _dot(h, aw_r[...])
        hr_r[...] = _dot(h, ar_r[...])

    full = pl.BlockSpec((C, C), lambda i: (0, 0))
    bias = pl.BlockSpec((1, C), lambda i: (0, 0))
    blk = pl.BlockSpec((BM, C), lambda i: (i, 0))
    return pl.pallas_call(
        body,
        grid=(N // BM,),
        in_specs=[blk, full, bias, full, bias, full, bias, full, full],
        out_specs=[blk, blk, blk],
        out_shape=[jax.ShapeDtypeStruct((N, C), jnp.float32)] * 3,
    )(x, W_ig, b_ig.reshape(1, C), W_og, b_og.reshape(1, C),
      W_in, b_in.reshape(1, C), arma_w, arma_root)


def _tc_scale(deg_parts, t_pre):
    N, C = t_pre.shape

    def body(dp_r, tp_r, tpo_r, dinv_r):
        deg = jnp.sum(dp_r[...], axis=0, keepdims=True)
        dinv = jnp.where(deg > 0.0,
                         lax.rsqrt(jnp.maximum(deg, 1e-12)), 0.0)
        dcol = jnp.transpose(dinv, (1, 0))
        tpo_r[...] = tp_r[...] * dcol
        dinv_r[...] = dcol

    return pl.pallas_call(
        body,
        out_shape=[jax.ShapeDtypeStruct((N, C), jnp.float32),
                   jax.ShapeDtypeStruct((N, 1), jnp.float32)],
    )(deg_parts, t_pre)


def _tc_post(parts, dinv2d, hr, og, W_gw, b_gw, W_out, b_out, arma_bias):
    N, C = hr.shape
    BM = 1000

    def body(p_r, dv_r, hr_r, og_r, wgw_r, bgw_r, wo_r, bo_r, ab_r, out_r):
        agg = (p_r[0] + p_r[1]) * dv_r[...]
        a = jnp.maximum(agg + hr_r[...] + ab_r[...], 0.0)
        g = _gelu(_dot_t(a, wgw_r[...]) + bgw_r[...])
        out_r[...] = og_r[...] * (_dot_t(g, wo_r[...]) + bo_r[...])

    full = pl.BlockSpec((C, C), lambda i: (0, 0))
    bias = pl.BlockSpec((1, C), lambda i: (0, 0))
    blk = pl.BlockSpec((BM, C), lambda i: (i, 0))
    return pl.pallas_call(
        body,
        grid=(N // BM,),
        in_specs=[pl.BlockSpec((2, BM, C), lambda i: (0, i, 0)),
                  pl.BlockSpec((BM, 1), lambda i: (i, 0)),
                  blk, blk, full, bias, full, bias, bias],
        out_specs=blk,
        out_shape=jax.ShapeDtypeStruct((N, C), jnp.float32),
    )(parts, dinv2d, hr, og, W_gw, b_gw.reshape(1, C),
      W_out, b_out.reshape(1, C), arma_bias.reshape(1, C))


_MESH = dict(core_axis_name="c", subcore_axis_name="s")


def _sc_degree(col, n_nodes):
    E = col.shape[0]
    mesh = plsc.VectorSubcoreMesh(**_MESH)
    NW = mesh.num_cores * mesh.num_subcores
    EPW = E // NW
    CH = 2000

    @functools.partial(
        pl.kernel,
        out_type=jax.ShapeDtypeStruct((NW, n_nodes), jnp.float32),
        mesh=mesh,
        compiler_params=pltpu.CompilerParams(needs_layout_passes=False),
        scratch_types=[pltpu.VMEM((CH,), jnp.int32),
                       pltpu.VMEM((n_nodes,), jnp.float32)],
    )
    def k(col_hbm, out_hbm, idx_v, deg_v):
        cid = lax.axis_index("c")
        sid = lax.axis_index("s")
        wid = sid * mesh.num_cores + cid
        z16 = jnp.zeros((16,), jnp.float32)
        one16 = jnp.ones((16,), jnp.float32)

        @pl.loop(0, n_nodes, step=16)
        def _(i):
            deg_v[pl.ds(i, 16)] = z16

        base = wid * EPW

        @pl.loop(0, EPW, step=CH)
        def _(off):
            pltpu.sync_copy(col_hbm.at[pl.ds(base + off, CH)], idx_v)

            @pl.loop(0, CH, step=16)
            def _(j):
                plsc.addupdate_scatter(deg_v, [idx_v[pl.ds(j, 16)]], one16)

        pltpu.sync_copy(deg_v, out_hbm.at[wid])

    return k(col)


def _sc_aggregate(tprime, row, col):
    N, C = tprime.shape
    E = row.shape[0]
    mesh = plsc.VectorSubcoreMesh(**_MESH)
    NC, NS = mesh.num_cores, mesh.num_subcores
    NW = NC * NS
    EPW = E // NW
    CH = 80
    ZR = 80

    @functools.partial(
        pl.kernel,
        out_type=jax.ShapeDtypeStruct((NC, N, C), jnp.float32),
        mesh=mesh,
        scratch_types=[pltpu.VMEM((CH,), jnp.int32),
                       pltpu.VMEM((CH,), jnp.int32),
                       pltpu.VMEM((CH, C), jnp.float32),
                       pltpu.VMEM((ZR, C), jnp.float32),
                       pltpu.VMEM_SHARED((N, C), jnp.float32)],
    )
    def k(t_hbm, row_hbm, col_hbm, out_hbm, ridx_v, cidx_v, rows_v,
          zero_v, agg_sh):
        cid = lax.axis_index("c")
        sid = lax.axis_index("s")
        wid = sid * NC + cid
        z16 = jnp.zeros((16,), jnp.float32)

        @pl.loop(0, ZR)
        def _(r):
            @pl.loop(0, C, step=16)
            def _(cc):
                zero_v[r, pl.ds(cc, 16)] = z16

        @pl.loop(sid * ZR, N, step=NS * ZR)
        def _(rr):
            pltpu.sync_copy(zero_v, agg_sh.at[pl.ds(rr, ZR)])

        plsc.subcore_barrier()

        base = wid * EPW

        @pl.loop(0, EPW, step=CH)
        def _(off):
            pltpu.sync_copy(row_hbm.at[pl.ds(base + off, CH)], ridx_v)
            pltpu.sync_copy(col_hbm.at[pl.ds(base + off, CH)], cidx_v)
            pltpu.sync_copy(t_hbm.at[ridx_v], rows_v)
            pltpu.sync_copy(rows_v, agg_sh.at[cidx_v], add=True)

        plsc.subcore_barrier()

        @pl.loop(sid * ZR, N, step=NS * ZR)
        def _(rr):
            pltpu.sync_copy(agg_sh.at[pl.ds(rr, ZR)],
                            out_hbm.at[cid, pl.ds(rr, ZR)])

    return k(tprime, row, col)


def kernel(x, edge_index, W_ig, b_ig, W_og, b_og, W_in, b_in, W_gw, b_gw,
           W_out, b_out, arma_w, arma_root, arma_bias):
    N, C = x.shape
    row = edge_index[0]
    col = edge_index[1]

    og, hr, t_pre = _tc_pre(x, W_ig, b_ig, W_og, b_og, W_in, b_in,
                            arma_w, arma_root)
    deg_parts = _sc_degree(col, N)
    tprime, dinv2d = _tc_scale(deg_parts, t_pre)
    parts = _sc_aggregate(tprime, row, col)
    return _tc_post(parts, dinv2d, hr, og, W_gw, b_gw, W_out, b_out,
                    arma_bias)


# speedup vs baseline: 38.5408x; 2.3297x over previous
import functools

import jax
import jax.numpy as jnp
from jax import lax
from jax.experimental import pallas as pl
from jax.experimental.pallas import tpu as pltpu
from jax.experimental.pallas import tpu_sc as plsc


def _gelu(v):
    return 0.5 * v * (1.0 + lax.erf(v * 0.7071067811865476))


def _sigmoid(v):
    return 1.0 / (1.0 + jnp.exp(-v))


def _dot_t(a, b):
    return lax.dot_general(a, b, (((1,), (1,)), ((), ())),
                           preferred_element_type=jnp.float32)


def _dot(a, b):
    return lax.dot_general(a, b, (((1,), (0,)), ((), ())),
                           preferred_element_type=jnp.float32)


def _tc_pre(x, W_ig, b_ig, W_og, b_og, W_in, b_in, arma_w, arma_root):
    N, C = x.shape
    BM = 1000

    def body(x_r, wig_r, big_r, wog_r, bog_r, win_r, bin_r, aw_r, ar_r,
             og_r, hr_r, tp_r):
        xb = x_r[...]
        ig = _sigmoid(_dot_t(xb, wig_r[...]) + big_r[...])
        og_r[...] = _sigmoid(_dot_t(xb, wog_r[...]) + bog_r[...])
        h = _gelu(_dot_t(ig * xb, win_r[...]) + bin_r[...])
        tp_r[...] = _dot(h, aw_r[...])
        hr_r[...] = _dot(h, ar_r[...])

    full = pl.BlockSpec((C, C), lambda i: (0, 0))
    bias = pl.BlockSpec((1, C), lambda i: (0, 0))
    blk = pl.BlockSpec((BM, C), lambda i: (i, 0))
    return pl.pallas_call(
        body,
        grid=(N // BM,),
        in_specs=[blk, full, bias, full, bias, full, bias, full, full],
        out_specs=[blk, blk, blk],
        out_shape=[jax.ShapeDtypeStruct((N, C), jnp.float32)] * 3,
    )(x, W_ig, b_ig.reshape(1, C), W_og, b_og.reshape(1, C),
      W_in, b_in.reshape(1, C), arma_w, arma_root)


def _tc_scale(deg_parts, t_pre):
    N, C = t_pre.shape

    def body(dp_r, tp_r, tpo_r, dinv_r):
        deg = jnp.sum(dp_r[...], axis=0, keepdims=True)
        dinv = jnp.where(deg > 0.0,
                         lax.rsqrt(jnp.maximum(deg, 1e-12)), 0.0)
        dcol = jnp.transpose(dinv, (1, 0))
        tpo_r[...] = tp_r[...] * dcol
        dinv_r[...] = dcol

    return pl.pallas_call(
        body,
        out_shape=[jax.ShapeDtypeStruct((N, C), jnp.float32),
                   jax.ShapeDtypeStruct((N, 1), jnp.float32)],
    )(deg_parts, t_pre)


def _tc_post(parts, dinv2d, hr, og, W_gw, b_gw, W_out, b_out, arma_bias):
    N, C = hr.shape
    BM = 1000

    def body(p_r, dv_r, hr_r, og_r, wgw_r, bgw_r, wo_r, bo_r, ab_r, out_r):
        agg = (p_r[0] + p_r[1]) * dv_r[...]
        a = jnp.maximum(agg + hr_r[...] + ab_r[...], 0.0)
        g = _gelu(_dot_t(a, wgw_r[...]) + bgw_r[...])
        out_r[...] = og_r[...] * (_dot_t(g, wo_r[...]) + bo_r[...])

    full = pl.BlockSpec((C, C), lambda i: (0, 0))
    bias = pl.BlockSpec((1, C), lambda i: (0, 0))
    blk = pl.BlockSpec((BM, C), lambda i: (i, 0))
    return pl.pallas_call(
        body,
        grid=(N // BM,),
        in_specs=[pl.BlockSpec((2, BM, C), lambda i: (0, i, 0)),
                  pl.BlockSpec((BM, 1), lambda i: (i, 0)),
                  blk, blk, full, bias, full, bias, bias],
        out_specs=blk,
        out_shape=jax.ShapeDtypeStruct((N, C), jnp.float32),
    )(parts, dinv2d, hr, og, W_gw, b_gw.reshape(1, C),
      W_out, b_out.reshape(1, C), arma_bias.reshape(1, C))


_MESH = dict(core_axis_name="c", subcore_axis_name="s")


def _sc_degree(col, n_nodes):
    E = col.shape[0]
    mesh = plsc.VectorSubcoreMesh(**_MESH)
    NW = mesh.num_cores * mesh.num_subcores
    EPW = E // NW
    CH = 2000

    @functools.partial(
        pl.kernel,
        out_type=jax.ShapeDtypeStruct((NW, n_nodes), jnp.float32),
        mesh=mesh,
        compiler_params=pltpu.CompilerParams(needs_layout_passes=False),
        scratch_types=[pltpu.VMEM((CH,), jnp.int32),
                       pltpu.VMEM((n_nodes,), jnp.float32)],
    )
    def k(col_hbm, out_hbm, idx_v, deg_v):
        cid = lax.axis_index("c")
        sid = lax.axis_index("s")
        wid = sid * mesh.num_cores + cid
        z16 = jnp.zeros((16,), jnp.float32)
        one16 = jnp.ones((16,), jnp.float32)

        @pl.loop(0, n_nodes, step=16)
        def _(i):
            deg_v[pl.ds(i, 16)] = z16

        base = wid * EPW

        @pl.loop(0, EPW, step=CH)
        def _(off):
            pltpu.sync_copy(col_hbm.at[pl.ds(base + off, CH)], idx_v)

            @pl.loop(0, CH, step=16)
            def _(j):
                plsc.addupdate_scatter(deg_v, [idx_v[pl.ds(j, 16)]], one16)

        pltpu.sync_copy(deg_v, out_hbm.at[wid])

    return k(col)


def _sc_aggregate(tprime, row, col):
    N, C = tprime.shape
    E = row.shape[0]
    mesh = plsc.VectorSubcoreMesh(**_MESH)
    NC, NS = mesh.num_cores, mesh.num_subcores
    NW = NC * NS
    EPW = E // NW
    CH = 80
    ZR = 80
    RB = 4
    NCH = EPW // CH
    MAIN = NCH - RB - 1
    MAIN -= MAIN % RB

    @functools.partial(
        pl.kernel,
        out_type=jax.ShapeDtypeStruct((NC, N, C), jnp.float32),
        mesh=mesh,
        scratch_types=([pltpu.VMEM((RB, CH), jnp.int32),
                        pltpu.VMEM((RB, CH), jnp.int32),
                        pltpu.VMEM((RB, CH, C), jnp.float32),
                        pltpu.VMEM_SHARED((N, C), jnp.float32)]
                       + [pltpu.SemaphoreType.DMA] * (2 * RB)),
    )
    def k(t_hbm, row_hbm, col_hbm, out_hbm, ridx, cidx, rows,
          agg_sh, *sems):
        isem = sems[:RB]
        gsem = sems[RB:]
        cid = lax.axis_index("c")
        sid = lax.axis_index("s")
        wid = sid * NC + cid
        z16 = jnp.zeros((16,), jnp.float32)

        @pl.loop(0, ZR)
        def _(r):
            @pl.loop(0, C, step=16)
            def _(cc):
                rows[0, r, pl.ds(cc, 16)] = z16

        @pl.loop(sid * ZR, N, step=NS * ZR)
        def _(rr):
            pltpu.sync_copy(rows.at[0], agg_sh.at[pl.ds(rr, ZR)])

        plsc.subcore_barrier()

        base = wid * EPW

        def idx_start(kk, b):
            off = base + kk * CH
            pltpu.async_copy(row_hbm.at[pl.ds(off, CH)], ridx.at[b],
                             isem[b])
            pltpu.async_copy(col_hbm.at[pl.ds(off, CH)], cidx.at[b],
                             isem[b])

        def idx_wait(b):
            pltpu.make_async_copy(row_hbm.at[pl.ds(0, CH)], ridx.at[b],
                                  isem[b]).wait()
            pltpu.make_async_copy(col_hbm.at[pl.ds(0, CH)], cidx.at[b],
                                  isem[b]).wait()

        def gath_start(b):
            pltpu.async_copy(t_hbm.at[ridx.at[b]], rows.at[b], gsem[b])

        def gath_wait(b):
            pltpu.make_async_copy(t_hbm.at[ridx.at[b]], rows.at[b],
                                  gsem[b]).wait()

        def scat_sync(b):
            pltpu.sync_copy(rows.at[b], agg_sh.at[cidx.at[b]], add=True)

        for j in range(RB):
            idx_start(j, j)
        for j in range(RB - 1):
            idx_wait(j)
            gath_start(j)

        @pl.loop(0, MAIN, step=RB)
        def _(outer):
            for r in range(RB):
                kk = outer + r
                b = r
                bg = (r + RB - 1) % RB
                gath_wait(b)
                scat_sync(b)
                idx_start(kk + RB, b)
                idx_wait(bg)
                gath_start(bg)

        for j in range(MAIN, NCH):
            b = j % RB
            bg = (j + RB - 1) % RB
            gath_wait(b)
            scat_sync(b)
            if j + RB < NCH:
                idx_start(j + RB, b)
            if j + RB - 1 < NCH:
                idx_wait(bg)
                gath_start(bg)

        plsc.subcore_barrier()

        @pl.loop(sid * ZR, N, step=NS * ZR)
        def _(rr):
            pltpu.sync_copy(agg_sh.at[pl.ds(rr, ZR)],
                            out_hbm.at[cid, pl.ds(rr, ZR)])

    return k(tprime, row, col)


def kernel(x, edge_index, W_ig, b_ig, W_og, b_og, W_in, b_in, W_gw, b_gw,
           W_out, b_out, arma_w, arma_root, arma_bias):
    N, C = x.shape
    row = edge_index[0]
    col = edge_index[1]

    og, hr, t_pre = _tc_pre(x, W_ig, b_ig, W_og, b_og, W_in, b_in,
                            arma_w, arma_root)
    deg_parts = _sc_degree(col, N)
    tprime, dinv2d = _tc_scale(deg_parts, t_pre)
    parts = _sc_aggregate(tprime, row, col)
    return _tc_post(parts, dinv2d, hr, og, W_gw, b_gw, W_out, b_out,
                    arma_bias)


# speedup vs baseline: 41.6024x; 1.0794x over previous
import functools

import jax
import jax.numpy as jnp
from jax import lax
from jax.experimental import pallas as pl
from jax.experimental.pallas import tpu as pltpu
from jax.experimental.pallas import tpu_sc as plsc


def _gelu(v):
    return 0.5 * v * (1.0 + lax.erf(v * 0.7071067811865476))


def _sigmoid(v):
    return 1.0 / (1.0 + jnp.exp(-v))


def _dot_t(a, b):
    return lax.dot_general(a, b, (((1,), (1,)), ((), ())),
                           preferred_element_type=jnp.float32)


def _dot(a, b):
    return lax.dot_general(a, b, (((1,), (0,)), ((), ())),
                           preferred_element_type=jnp.float32)


def _tc_pre(x, W_ig, b_ig, W_og, b_og, W_in, b_in, arma_w, arma_root):
    N, C = x.shape
    BM = 1000

    def body(x_r, wig_r, big_r, wog_r, bog_r, win_r, bin_r, aw_r, ar_r,
             og_r, hr_r, tp_r):
        xb = x_r[...]
        ig = _sigmoid(_dot_t(xb, wig_r[...]) + big_r[...])
        og_r[...] = _sigmoid(_dot_t(xb, wog_r[...]) + bog_r[...])
        h = _gelu(_dot_t(ig * xb, win_r[...]) + bin_r[...])
        tp_r[...] = _dot(h, aw_r[...])
        hr_r[...] = _dot(h, ar_r[...])

    full = pl.BlockSpec((C, C), lambda i: (0, 0))
    bias = pl.BlockSpec((1, C), lambda i: (0, 0))
    blk = pl.BlockSpec((BM, C), lambda i: (i, 0))
    return pl.pallas_call(
        body,
        grid=(N // BM,),
        in_specs=[blk, full, bias, full, bias, full, bias, full, full],
        out_specs=[blk, blk, blk],
        out_shape=[jax.ShapeDtypeStruct((N, C), jnp.float32)] * 3,
    )(x, W_ig, b_ig.reshape(1, C), W_og, b_og.reshape(1, C),
      W_in, b_in.reshape(1, C), arma_w, arma_root)


def _tc_scale(deg_parts, t_pre):
    N, C = t_pre.shape

    def body(dp_r, tp_r, tpo_r, dinv_r):
        deg = jnp.sum(dp_r[...], axis=0, keepdims=True)
        dinv = jnp.where(deg > 0.0,
                         lax.rsqrt(jnp.maximum(deg, 1e-12)), 0.0)
        dcol = jnp.transpose(dinv, (1, 0))
        tpo_r[...] = tp_r[...] * dcol
        dinv_r[...] = dcol

    return pl.pallas_call(
        body,
        out_shape=[jax.ShapeDtypeStruct((N, C), jnp.float32),
                   jax.ShapeDtypeStruct((N, 1), jnp.float32)],
    )(deg_parts, t_pre)


def _tc_post(parts, dinv2d, hr, og, W_gw, b_gw, W_out, b_out, arma_bias):
    N, C = hr.shape
    BM = 1000

    def body(p_r, dv_r, hr_r, og_r, wgw_r, bgw_r, wo_r, bo_r, ab_r, out_r):
        agg = (p_r[0] + p_r[1]) * dv_r[...]
        a = jnp.maximum(agg + hr_r[...] + ab_r[...], 0.0)
        g = _gelu(_dot_t(a, wgw_r[...]) + bgw_r[...])
        out_r[...] = og_r[...] * (_dot_t(g, wo_r[...]) + bo_r[...])

    full = pl.BlockSpec((C, C), lambda i: (0, 0))
    bias = pl.BlockSpec((1, C), lambda i: (0, 0))
    blk = pl.BlockSpec((BM, C), lambda i: (i, 0))
    return pl.pallas_call(
        body,
        grid=(N // BM,),
        in_specs=[pl.BlockSpec((2, BM, C), lambda i: (0, i, 0)),
                  pl.BlockSpec((BM, 1), lambda i: (i, 0)),
                  blk, blk, full, bias, full, bias, bias],
        out_specs=blk,
        out_shape=jax.ShapeDtypeStruct((N, C), jnp.float32),
    )(parts, dinv2d, hr, og, W_gw, b_gw.reshape(1, C),
      W_out, b_out.reshape(1, C), arma_bias.reshape(1, C))


_MESH = dict(core_axis_name="c", subcore_axis_name="s")


def _sc_degree(ei_flat, n_nodes):
    E = ei_flat.shape[0] // 2
    mesh = plsc.VectorSubcoreMesh(**_MESH)
    NW = mesh.num_cores * mesh.num_subcores
    EPW = E // NW
    CH = 2000

    @functools.partial(
        pl.kernel,
        out_type=jax.ShapeDtypeStruct((NW, n_nodes), jnp.float32),
        mesh=mesh,
        compiler_params=pltpu.CompilerParams(needs_layout_passes=False),
        scratch_types=[pltpu.VMEM((CH,), jnp.int32),
                       pltpu.VMEM((n_nodes,), jnp.float32)],
    )
    def k(ei_hbm, out_hbm, idx_v, deg_v):
        cid = lax.axis_index("c")
        sid = lax.axis_index("s")
        wid = sid * mesh.num_cores + cid
        z16 = jnp.zeros((16,), jnp.float32)
        one16 = jnp.ones((16,), jnp.float32)

        @pl.loop(0, n_nodes, step=16)
        def _(i):
            deg_v[pl.ds(i, 16)] = z16

        base = wid * EPW

        @pl.loop(0, EPW, step=CH)
        def _(off):
            pltpu.sync_copy(ei_hbm.at[pl.ds(E + base + off, CH)], idx_v)

            @pl.loop(0, CH, step=16)
            def _(j):
                plsc.addupdate_scatter(deg_v, [idx_v[pl.ds(j, 16)]], one16)

        pltpu.sync_copy(deg_v, out_hbm.at[wid])

    return k(ei_flat)


def _sc_aggregate(tprime, ei_flat):
    N, C = tprime.shape
    E = ei_flat.shape[0] // 2
    mesh = plsc.VectorSubcoreMesh(**_MESH)
    NC, NS = mesh.num_cores, mesh.num_subcores
    NW = NC * NS
    EPW = E // NW
    CH = 80
    ZR = 80
    RB = 4
    NCH = EPW // CH
    MAIN = NCH - RB - 1
    MAIN -= MAIN % RB

    @functools.partial(
        pl.kernel,
        out_type=jax.ShapeDtypeStruct((NC, N, C), jnp.float32),
        mesh=mesh,
        scratch_types=([pltpu.VMEM((RB, CH), jnp.int32),
                        pltpu.VMEM((RB, CH), jnp.int32),
                        pltpu.VMEM((RB, CH, C), jnp.float32),
                        pltpu.VMEM_SHARED((N, C), jnp.float32)]
                       + [pltpu.SemaphoreType.DMA] * (2 * RB)),
    )
    def k(t_hbm, ei_hbm, out_hbm, ridx, cidx, rows,
          agg_sh, *sems):
        isem = sems[:RB]
        gsem = sems[RB:]
        cid = lax.axis_index("c")
        sid = lax.axis_index("s")
        wid = sid * NC + cid
        z16 = jnp.zeros((16,), jnp.float32)

        @pl.loop(0, ZR)
        def _(r):
            @pl.loop(0, C, step=16)
            def _(cc):
                rows[0, r, pl.ds(cc, 16)] = z16

        @pl.loop(sid * ZR, N, step=NS * ZR)
        def _(rr):
            pltpu.sync_copy(rows.at[0], agg_sh.at[pl.ds(rr, ZR)])

        plsc.subcore_barrier()

        base = wid * EPW

        def idx_start(kk, b):
            off = base + kk * CH
            pltpu.async_copy(ei_hbm.at[pl.ds(off, CH)], ridx.at[b],
                             isem[b])
            pltpu.async_copy(ei_hbm.at[pl.ds(E + off, CH)], cidx.at[b],
                             isem[b])

        def idx_wait(b):
            pltpu.make_async_copy(ei_hbm.at[pl.ds(0, CH)], ridx.at[b],
                                  isem[b]).wait()
            pltpu.make_async_copy(ei_hbm.at[pl.ds(0, CH)], cidx.at[b],
                                  isem[b]).wait()

        def gath_start(b):
            pltpu.async_copy(t_hbm.at[ridx.at[b]], rows.at[b], gsem[b])

        def gath_wait(b):
            pltpu.make_async_copy(t_hbm.at[ridx.at[b]], rows.at[b],
                                  gsem[b]).wait()

        def scat_sync(b):
            pltpu.sync_copy(rows.at[b], agg_sh.at[cidx.at[b]], add=True)

        for j in range(RB):
            idx_start(j, j)
        for j in range(RB - 1):
            idx_wait(j)
            gath_start(j)

        @pl.loop(0, MAIN, step=RB)
        def _(outer):
            for r in range(RB):
                kk = outer + r
                b = r
                bg = (r + RB - 1) % RB
                gath_wait(b)
                scat_sync(b)
                idx_start(kk + RB, b)
                idx_wait(bg)
                gath_start(bg)

        for j in range(MAIN, NCH):
            b = j % RB
            bg = (j + RB - 1) % RB
            gath_wait(b)
            scat_sync(b)
            if j + RB < NCH:
                idx_start(j + RB, b)
            if j + RB - 1 < NCH:
                idx_wait(bg)
                gath_start(bg)

        plsc.subcore_barrier()

        @pl.loop(sid * ZR, N, step=NS * ZR)
        def _(rr):
            pltpu.sync_copy(agg_sh.at[pl.ds(rr, ZR)],
                            out_hbm.at[cid, pl.ds(rr, ZR)])

    return k(tprime, ei_flat)


def kernel(x, edge_index, W_ig, b_ig, W_og, b_og, W_in, b_in, W_gw, b_gw,
           W_out, b_out, arma_w, arma_root, arma_bias):
    N, C = x.shape
    ei_flat = edge_index.reshape(-1)

    og, hr, t_pre = _tc_pre(x, W_ig, b_ig, W_og, b_og, W_in, b_in,
                            arma_w, arma_root)
    deg_parts = _sc_degree(ei_flat, N)
    tprime, dinv2d = _tc_scale(deg_parts, t_pre)
    parts = _sc_aggregate(tprime, ei_flat)
    return _tc_post(parts, dinv2d, hr, og, W_gw, b_gw, W_out, b_out,
                    arma_bias)


# speedup vs baseline: 41.6770x; 1.0018x over previous
import functools

import jax
import jax.numpy as jnp
from jax import lax
from jax.experimental import pallas as pl
from jax.experimental.pallas import tpu as pltpu
from jax.experimental.pallas import tpu_sc as plsc


def _gelu(v):
    return 0.5 * v * (1.0 + lax.erf(v * 0.7071067811865476))


def _sigmoid(v):
    return 1.0 / (1.0 + jnp.exp(-v))


def _dot_t(a, b):
    return lax.dot_general(a, b, (((1,), (1,)), ((), ())),
                           preferred_element_type=jnp.float32)


def _dot(a, b):
    return lax.dot_general(a, b, (((1,), (0,)), ((), ())),
                           preferred_element_type=jnp.float32)


def _tc_pre(x, W_ig, b_ig, W_og, b_og, W_in, b_in, arma_w, arma_root):
    N, C = x.shape
    BM = 1000

    def body(x_r, wig_r, big_r, wog_r, bog_r, win_r, bin_r, aw_r, ar_r,
             og_r, hr_r, tp_r):
        xb = x_r[...]
        ig = _sigmoid(_dot_t(xb, wig_r[...]) + big_r[...])
        og_r[...] = _sigmoid(_dot_t(xb, wog_r[...]) + bog_r[...])
        h = _gelu(_dot_t(ig * xb, win_r[...]) + bin_r[...])
        tp_r[...] = _dot(h, aw_r[...])
        hr_r[...] = _dot(h, ar_r[...])

    full = pl.BlockSpec((C, C), lambda i: (0, 0))
    bias = pl.BlockSpec((1, C), lambda i: (0, 0))
    blk = pl.BlockSpec((BM, C), lambda i: (i, 0))
    return pl.pallas_call(
        body,
        grid=(N // BM,),
        in_specs=[blk, full, bias, full, bias, full, bias, full, full],
        out_specs=[blk, blk, blk],
        out_shape=[jax.ShapeDtypeStruct((N, C), jnp.float32)] * 3,
    )(x, W_ig, b_ig.reshape(1, C), W_og, b_og.reshape(1, C),
      W_in, b_in.reshape(1, C), arma_w, arma_root)


def _tc_scale(deg_parts, t_pre):
    N, C = t_pre.shape

    def body(dp_r, tp_r, tpo_r, dinv_r):
        deg = jnp.sum(dp_r[...], axis=0, keepdims=True)
        dinv = jnp.where(deg > 0.0,
                         lax.rsqrt(jnp.maximum(deg, 1e-12)), 0.0)
        dcol = jnp.transpose(dinv, (1, 0))
        tpo_r[...] = tp_r[...] * dcol
        dinv_r[...] = dcol

    return pl.pallas_call(
        body,
        out_shape=[jax.ShapeDtypeStruct((N, C), jnp.float32),
                   jax.ShapeDtypeStruct((N, 1), jnp.float32)],
    )(deg_parts, t_pre)


def _tc_post(parts, dinv2d, hr, og, W_gw, b_gw, W_out, b_out, arma_bias):
    N, C = hr.shape
    BM = 1000

    def body(p_r, dv_r, hr_r, og_r, wgw_r, bgw_r, wo_r, bo_r, ab_r, out_r):
        agg = (p_r[0] + p_r[1]) * dv_r[...]
        a = jnp.maximum(agg + hr_r[...] + ab_r[...], 0.0)
        g = _gelu(_dot_t(a, wgw_r[...]) + bgw_r[...])
        out_r[...] = og_r[...] * (_dot_t(g, wo_r[...]) + bo_r[...])

    full = pl.BlockSpec((C, C), lambda i: (0, 0))
    bias = pl.BlockSpec((1, C), lambda i: (0, 0))
    blk = pl.BlockSpec((BM, C), lambda i: (i, 0))
    return pl.pallas_call(
        body,
        grid=(N // BM,),
        in_specs=[pl.BlockSpec((2, BM, C), lambda i: (0, i, 0)),
                  pl.BlockSpec((BM, 1), lambda i: (i, 0)),
                  blk, blk, full, bias, full, bias, bias],
        out_specs=blk,
        out_shape=jax.ShapeDtypeStruct((N, C), jnp.float32),
    )(parts, dinv2d, hr, og, W_gw, b_gw.reshape(1, C),
      W_out, b_out.reshape(1, C), arma_bias.reshape(1, C))


_MESH = dict(core_axis_name="c", subcore_axis_name="s")


def _sc_degree(ei_flat, n_nodes):
    E = ei_flat.shape[0] // 2
    mesh = plsc.VectorSubcoreMesh(**_MESH)
    NW = mesh.num_cores * mesh.num_subcores
    EPW = E // NW
    CH = 2000

    @functools.partial(
        pl.kernel,
        out_type=jax.ShapeDtypeStruct((NW, n_nodes), jnp.float32),
        mesh=mesh,
        compiler_params=pltpu.CompilerParams(needs_layout_passes=False),
        scratch_types=[pltpu.VMEM((CH,), jnp.int32),
                       pltpu.VMEM((n_nodes,), jnp.float32)],
    )
    def k(ei_hbm, out_hbm, idx_v, deg_v):
        cid = lax.axis_index("c")
        sid = lax.axis_index("s")
        wid = sid * mesh.num_cores + cid
        z16 = jnp.zeros((16,), jnp.float32)
        one16 = jnp.ones((16,), jnp.float32)

        @pl.loop(0, n_nodes, step=16)
        def _(i):
            deg_v[pl.ds(i, 16)] = z16

        base = wid * EPW

        @pl.loop(0, EPW, step=CH)
        def _(off):
            pltpu.sync_copy(ei_hbm.at[pl.ds(E + base + off, CH)], idx_v)

            @pl.loop(0, CH, step=16)
            def _(j):
                plsc.addupdate_scatter(deg_v, [idx_v[pl.ds(j, 16)]], one16)

        pltpu.sync_copy(deg_v, out_hbm.at[wid])

    return k(ei_flat)


def _sc_aggregate(tprime, ei_flat):
    N, C = tprime.shape
    E = ei_flat.shape[0] // 2
    mesh = plsc.VectorSubcoreMesh(**_MESH)
    NC, NS = mesh.num_cores, mesh.num_subcores
    NW = NC * NS
    EPW = E // NW
    CH = 80
    ZR = 80
    RB = 4
    IR = 2 * RB
    NCH = EPW // CH
    WARM = RB
    MAIN = NCH - WARM - RB - 1
    MAIN -= MAIN % IR
    MAIN += WARM

    @functools.partial(
        pl.kernel,
        out_type=jax.ShapeDtypeStruct((NC, N, C), jnp.float32),
        mesh=mesh,
        scratch_types=([pltpu.VMEM((IR, CH), jnp.int32),
                        pltpu.VMEM((IR, CH), jnp.int32),
                        pltpu.VMEM((RB, CH, C), jnp.float32),
                        pltpu.VMEM_SHARED((N, C), jnp.float32)]
                       + [pltpu.SemaphoreType.DMA] * (IR + 2 * RB)),
    )
    def k(t_hbm, ei_hbm, out_hbm, ridx, cidx, rows,
          agg_sh, *sems):
        isem = sems[:IR]
        gsem = sems[IR:IR + RB]
        ssem = sems[IR + RB:]
        cid = lax.axis_index("c")
        sid = lax.axis_index("s")
        wid = sid * NC + cid
        z16 = jnp.zeros((16,), jnp.float32)

        @pl.loop(0, ZR)
        def _(r):
            @pl.loop(0, C, step=16)
            def _(cc):
                rows[0, r, pl.ds(cc, 16)] = z16

        @pl.loop(sid * ZR, N, step=NS * ZR)
        def _(rr):
            pltpu.sync_copy(rows.at[0], agg_sh.at[pl.ds(rr, ZR)])

        plsc.subcore_barrier()

        base = wid * EPW

        def idx_start(kk, isl):
            off = base + kk * CH
            pltpu.async_copy(ei_hbm.at[pl.ds(off, CH)], ridx.at[isl],
                             isem[isl])
            pltpu.async_copy(ei_hbm.at[pl.ds(E + off, CH)], cidx.at[isl],
                             isem[isl])

        def idx_wait(isl):
            pltpu.make_async_copy(ei_hbm.at[pl.ds(0, CH)], ridx.at[isl],
                                  isem[isl]).wait()
            pltpu.make_async_copy(ei_hbm.at[pl.ds(0, CH)], cidx.at[isl],
                                  isem[isl]).wait()

        def gath_start(isl, b):
            pltpu.async_copy(t_hbm.at[ridx.at[isl]], rows.at[b], gsem[b])

        def gath_wait(isl, b):
            pltpu.make_async_copy(t_hbm.at[ridx.at[isl]], rows.at[b],
                                  gsem[b]).wait()

        def scat_start(isl, b):
            pltpu.async_copy(rows.at[b], agg_sh.at[cidx.at[isl]],
                             ssem[b], add=True)

        def scat_wait(isl, b):
            pltpu.make_async_copy(rows.at[b], agg_sh.at[cidx.at[isl]],
                                  ssem[b]).wait()

        for j in range(RB):
            idx_start(j, j)
        for j in range(RB - 1):
            idx_wait(j)
            gath_start(j, j)

        def body(kk, r, tail=False):
            b = r % RB
            isl = r % IR
            gath_wait(isl, b)
            scat_start(isl, b)
            if not isinstance(kk, int) or kk >= 1:
                scat_wait((r - 1) % IR, (r - 1) % RB)
            if not tail or r + RB - 1 < NCH:
                idx_wait((r + RB - 1) % IR)
                gath_start((r + RB - 1) % IR, (r + RB - 1) % RB)
            if not tail or r + RB < NCH:
                idx_start(kk + RB, (r + RB) % IR)

        for j in range(WARM):
            body(j, j)

        @pl.loop(WARM, MAIN, step=IR)
        def _(outer):
            for r in range(IR):
                body(outer + r, WARM + r)

        for j in range(MAIN, NCH):
            body(j, j, tail=True)
        scat_wait((NCH - 1) % IR, (NCH - 1) % RB)

        plsc.subcore_barrier()

        @pl.loop(sid * ZR, N, step=NS * ZR)
        def _(rr):
            pltpu.sync_copy(agg_sh.at[pl.ds(rr, ZR)],
                            out_hbm.at[cid, pl.ds(rr, ZR)])

    return k(tprime, ei_flat)


def kernel(x, edge_index, W_ig, b_ig, W_og, b_og, W_in, b_in, W_gw, b_gw,
           W_out, b_out, arma_w, arma_root, arma_bias):
    N, C = x.shape
    ei_flat = edge_index.reshape(-1)

    og, hr, t_pre = _tc_pre(x, W_ig, b_ig, W_og, b_og, W_in, b_in,
                            arma_w, arma_root)
    deg_parts = _sc_degree(ei_flat, N)
    tprime, dinv2d = _tc_scale(deg_parts, t_pre)
    parts = _sc_aggregate(tprime, ei_flat)
    return _tc_post(parts, dinv2d, hr, og, W_gw, b_gw, W_out, b_out,
                    arma_bias)


# speedup vs baseline: 41.6825x; 1.0001x over previous
import functools

import jax
import jax.numpy as jnp
from jax import lax
from jax.experimental import pallas as pl
from jax.experimental.pallas import tpu as pltpu
from jax.experimental.pallas import tpu_sc as plsc


def _gelu(v):
    return 0.5 * v * (1.0 + lax.erf(v * 0.7071067811865476))


def _sigmoid(v):
    return 1.0 / (1.0 + jnp.exp(-v))


def _dot_t(a, b):
    return lax.dot_general(a.astype(jnp.bfloat16), b.astype(jnp.bfloat16),
                           (((1,), (1,)), ((), ())),
                           preferred_element_type=jnp.float32)


def _dot(a, b):
    return lax.dot_general(a.astype(jnp.bfloat16), b.astype(jnp.bfloat16),
                           (((1,), (0,)), ((), ())),
                           preferred_element_type=jnp.float32)


def _tc_pre(x, W_ig, b_ig, W_og, b_og, W_in, b_in, arma_w, arma_root):
    N, C = x.shape
    BM = 1000

    def body(x_r, wig_r, big_r, wog_r, bog_r, win_r, bin_r, aw_r, ar_r,
             og_r, hr_r, tp_r):
        xb = x_r[...]
        ig = _sigmoid(_dot_t(xb, wig_r[...]) + big_r[...])
        og_r[...] = _sigmoid(_dot_t(xb, wog_r[...]) + bog_r[...])
        h = _gelu(_dot_t(ig * xb, win_r[...]) + bin_r[...])
        tp_r[...] = _dot(h, aw_r[...])
        hr_r[...] = _dot(h, ar_r[...])

    full = pl.BlockSpec((C, C), lambda i: (0, 0))
    bias = pl.BlockSpec((1, C), lambda i: (0, 0))
    blk = pl.BlockSpec((BM, C), lambda i: (i, 0))
    return pl.pallas_call(
        body,
        grid=(N // BM,),
        in_specs=[blk, full, bias, full, bias, full, bias, full, full],
        out_specs=[blk, blk, blk],
        out_shape=[jax.ShapeDtypeStruct((N, C), jnp.float32)] * 3,
    )(x, W_ig, b_ig.reshape(1, C), W_og, b_og.reshape(1, C),
      W_in, b_in.reshape(1, C), arma_w, arma_root)


def _tc_scale(deg_parts, t_pre):
    N, C = t_pre.shape

    def body(dp_r, tp_r, tpo_r, dinv_r):
        deg = jnp.sum(dp_r[...], axis=0, keepdims=True)
        dinv = jnp.where(deg > 0.0,
                         lax.rsqrt(jnp.maximum(deg, 1e-12)), 0.0)
        dcol = jnp.transpose(dinv, (1, 0))
        tpo_r[...] = tp_r[...] * dcol
        dinv_r[...] = dcol

    return pl.pallas_call(
        body,
        out_shape=[jax.ShapeDtypeStruct((N, C), jnp.float32),
                   jax.ShapeDtypeStruct((N, 1), jnp.float32)],
    )(deg_parts, t_pre)


def _tc_post(parts, dinv2d, hr, og, W_gw, b_gw, W_out, b_out, arma_bias):
    N, C = hr.shape
    BM = 1000

    def body(p_r, dv_r, hr_r, og_r, wgw_r, bgw_r, wo_r, bo_r, ab_r, out_r):
        agg = (p_r[0] + p_r[1]) * dv_r[...]
        a = jnp.maximum(agg + hr_r[...] + ab_r[...], 0.0)
        g = _gelu(_dot_t(a, wgw_r[...]) + bgw_r[...])
        out_r[...] = og_r[...] * (_dot_t(g, wo_r[...]) + bo_r[...])

    full = pl.BlockSpec((C, C), lambda i: (0, 0))
    bias = pl.BlockSpec((1, C), lambda i: (0, 0))
    blk = pl.BlockSpec((BM, C), lambda i: (i, 0))
    return pl.pallas_call(
        body,
        grid=(N // BM,),
        in_specs=[pl.BlockSpec((2, BM, C), lambda i: (0, i, 0)),
                  pl.BlockSpec((BM, 1), lambda i: (i, 0)),
                  blk, blk, full, bias, full, bias, bias],
        out_specs=blk,
        out_shape=jax.ShapeDtypeStruct((N, C), jnp.float32),
    )(parts, dinv2d, hr, og, W_gw, b_gw.reshape(1, C),
      W_out, b_out.reshape(1, C), arma_bias.reshape(1, C))


_MESH = dict(core_axis_name="c", subcore_axis_name="s")


def _sc_degree(ei_flat, n_nodes):
    E = ei_flat.shape[0] // 2
    mesh = plsc.VectorSubcoreMesh(**_MESH)
    NW = mesh.num_cores * mesh.num_subcores
    EPW = E // NW
    CH = 2000

    @functools.partial(
        pl.kernel,
        out_type=jax.ShapeDtypeStruct((NW, n_nodes), jnp.float32),
        mesh=mesh,
        compiler_params=pltpu.CompilerParams(needs_layout_passes=False),
        scratch_types=[pltpu.VMEM((CH,), jnp.int32),
                       pltpu.VMEM((n_nodes,), jnp.float32)],
    )
    def k(ei_hbm, out_hbm, idx_v, deg_v):
        cid = lax.axis_index("c")
        sid = lax.axis_index("s")
        wid = sid * mesh.num_cores + cid
        z16 = jnp.zeros((16,), jnp.float32)
        one16 = jnp.ones((16,), jnp.float32)

        @pl.loop(0, n_nodes, step=16)
        def _(i):
            deg_v[pl.ds(i, 16)] = z16

        base = wid * EPW

        @pl.loop(0, EPW, step=CH)
        def _(off):
            pltpu.sync_copy(ei_hbm.at[pl.ds(E + base + off, CH)], idx_v)

            @pl.loop(0, CH, step=16)
            def _(j):
                plsc.addupdate_scatter(deg_v, [idx_v[pl.ds(j, 16)]], one16)

        pltpu.sync_copy(deg_v, out_hbm.at[wid])

    return k(ei_flat)


def _sc_aggregate(tprime, ei_flat):
    N, C = tprime.shape
    E = ei_flat.shape[0] // 2
    mesh = plsc.VectorSubcoreMesh(**_MESH)
    NC, NS = mesh.num_cores, mesh.num_subcores
    NW = NC * NS
    EPW = E // NW
    CH = 80
    ZR = 80
    RB = 4
    IR = 2 * RB
    NCH = EPW // CH
    WARM = RB
    MAIN = NCH - WARM - RB - 1
    MAIN -= MAIN % IR
    MAIN += WARM

    @functools.partial(
        pl.kernel,
        out_type=jax.ShapeDtypeStruct((NC, N, C), jnp.float32),
        mesh=mesh,
        scratch_types=([pltpu.VMEM((IR, CH), jnp.int32),
                        pltpu.VMEM((IR, CH), jnp.int32),
                        pltpu.VMEM((RB, CH, C), jnp.float32),
                        pltpu.VMEM_SHARED((N, C), jnp.float32)]
                       + [pltpu.SemaphoreType.DMA] * (IR + 2 * RB)),
    )
    def k(t_hbm, ei_hbm, out_hbm, ridx, cidx, rows,
          agg_sh, *sems):
        isem = sems[:IR]
        gsem = sems[IR:IR + RB]
        ssem = sems[IR + RB:]
        cid = lax.axis_index("c")
        sid = lax.axis_index("s")
        wid = sid * NC + cid
        z16 = jnp.zeros((16,), jnp.float32)

        @pl.loop(0, ZR)
        def _(r):
            @pl.loop(0, C, step=16)
            def _(cc):
                rows[0, r, pl.ds(cc, 16)] = z16

        @pl.loop(sid * ZR, N, step=NS * ZR)
        def _(rr):
            pltpu.sync_copy(rows.at[0], agg_sh.at[pl.ds(rr, ZR)])

        plsc.subcore_barrier()

        base = wid * EPW

        def idx_start(kk, isl):
            off = base + kk * CH
            pltpu.async_copy(ei_hbm.at[pl.ds(off, CH)], ridx.at[isl],
                             isem[isl])
            pltpu.async_copy(ei_hbm.at[pl.ds(E + off, CH)], cidx.at[isl],
                             isem[isl])

        def idx_wait(isl):
            pltpu.make_async_copy(ei_hbm.at[pl.ds(0, CH)], ridx.at[isl],
                                  isem[isl]).wait()
            pltpu.make_async_copy(ei_hbm.at[pl.ds(0, CH)], cidx.at[isl],
                                  isem[isl]).wait()

        def gath_start(isl, b):
            pltpu.async_copy(t_hbm.at[ridx.at[isl]], rows.at[b], gsem[b])

        def gath_wait(isl, b):
            pltpu.make_async_copy(t_hbm.at[ridx.at[isl]], rows.at[b],
                                  gsem[b]).wait()

        def scat_start(isl, b):
            pltpu.async_copy(rows.at[b], agg_sh.at[cidx.at[isl]],
                             ssem[b], add=True)

        def scat_wait(isl, b):
            pltpu.make_async_copy(rows.at[b], agg_sh.at[cidx.at[isl]],
                                  ssem[b]).wait()

        for j in range(RB):
            idx_start(j, j)
        for j in range(RB - 1):
            idx_wait(j)
            gath_start(j, j)

        def body(kk, r, tail=False):
            b = r % RB
            isl = r % IR
            gath_wait(isl, b)
            scat_start(isl, b)
            if not isinstance(kk, int) or kk >= 1:
                scat_wait((r - 1) % IR, (r - 1) % RB)
            if not tail or r + RB - 1 < NCH:
                idx_wait((r + RB - 1) % IR)
                gath_start((r + RB - 1) % IR, (r + RB - 1) % RB)
            if not tail or r + RB < NCH:
                idx_start(kk + RB, (r + RB) % IR)

        for j in range(WARM):
            body(j, j)

        @pl.loop(WARM, MAIN, step=IR)
        def _(outer):
            for r in range(IR):
                body(outer + r, WARM + r)

        for j in range(MAIN, NCH):
            body(j, j, tail=True)
        scat_wait((NCH - 1) % IR, (NCH - 1) % RB)

        plsc.subcore_barrier()

        @pl.loop(sid * ZR, N, step=NS * ZR)
        def _(rr):
            pltpu.sync_copy(agg_sh.at[pl.ds(rr, ZR)],
                            out_hbm.at[cid, pl.ds(rr, ZR)])

    return k(tprime, ei_flat)


def kernel(x, edge_index, W_ig, b_ig, W_og, b_og, W_in, b_in, W_gw, b_gw,
           W_out, b_out, arma_w, arma_root, arma_bias):
    N, C = x.shape
    ei_flat = edge_index.reshape(-1)

    og, hr, t_pre = _tc_pre(x, W_ig, b_ig, W_og, b_og, W_in, b_in,
                            arma_w, arma_root)
    deg_parts = _sc_degree(ei_flat, N)
    tprime, dinv2d = _tc_scale(deg_parts, t_pre)
    parts = _sc_aggregate(tprime, ei_flat)
    return _tc_post(parts, dinv2d, hr, og, W_gw, b_gw, W_out, b_out,
                    arma_bias)
